# baseline (speedup 1.0000x reference)
import functools

import jax
import jax.numpy as jnp
import numpy as np
from jax import lax
from jax.experimental import pallas as pl
from jax.experimental.pallas import tpu as pltpu
from jax.experimental.pallas import tpu_sc as plsc

SEQ_LEN = 4096
D = 768
LANES = 16
NUM_WINDOWS = 21
WIN = 203
NC, NS = 2, 16
NW = NC * NS
NUNIT = 3


def _window_ranges():
    tokens_per_frame = SEQ_LEN / 81
    half = int(tokens_per_frame * 4 / 2)
    pos = []
    for i in range(NUM_WINDOWS):
        if i == 0:
            pos.append(0)
        else:
            st = tokens_per_frame * ((i - 1) * 4 + 1)
            en = tokens_per_frame * (i * 4 + 1)
            pos.append(int((st + en) / 2) - 1)
    ranges = [[p - half, p + half] for p in pos]
    ranges[0] = [-(half * 2 - ranges[1][0]), ranges[1][0]]
    return ranges


_RANGES = _window_ranges()
_K_LENS = []
for _s, _e in _RANGES:
    _vs, _ve = max(_s, 0), min(_e, SEQ_LEN - 1)
    _K_LENS.append(_ve - _vs + 1 if _vs <= _ve else 0)
_KLEN0 = _K_LENS[0]
_STARTS = [max(_s, 0) for _s, _ in _RANGES]

(CLS_G72A, CLS_G72B, CLS_G56, CLS_G3, CLS_M, CLS_Z16, CLS_Z72, CLS_Z56,
 CLS_Z3, CLS_NOP) = range(10)
_G_GEOM = {CLS_G72A: (72, 0), CLS_G72B: (72, 72), CLS_G56: (56, 144),
           CLS_G3: (3, 200)}
_Z_GEOM = {CLS_Z16: (16, 56), CLS_Z72: (72, 72), CLS_Z56: (56, 144),
           CLS_Z3: (3, 200)}

_UNITS = []
for _w in range(1, NUM_WINDOWS):
    _sw = _STARTS[_w]
    _lin = _sw % 8 == 0
    for _cls, (_n, _do) in _G_GEOM.items():
        _UNITS.append((_cls + (CLS_NOP + 1 if _lin else 0), _w, _sw + _do))
_UNITS.append((CLS_M, 0, 0))
for _cls in (CLS_Z16, CLS_Z72, CLS_Z56, CLS_Z3):
    _UNITS.append((_cls, 0, 0))
assert len(_UNITS) == 85

_COST = {CLS_G72A: 92, CLS_G72B: 92, CLS_G56: 76, CLS_G3: 23, CLS_M: 84,
         CLS_Z16: 36, CLS_Z72: 92, CLS_Z56: 76, CLS_Z3: 23,
         CLS_NOP + 1: 40, CLS_NOP + 2: 40, CLS_NOP + 3: 32, CLS_NOP + 4: 12}

_WORKER_UNITS = [[] for _ in range(NW)]
_LOADS = [0.0] * NW
for _u in sorted(_UNITS, key=lambda u: -_COST[u[0]]):
    _open = [i for i in range(NW) if len(_WORKER_UNITS[i]) < NUNIT]
    _k = min(_open, key=lambda i: (_LOADS[i], len(_WORKER_UNITS[i])))
    _WORKER_UNITS[_k].append(_u)
    _LOADS[_k] += _COST[_u[0]]

_TBL = np.zeros((NW, NUNIT, 16), np.int32)
_TBL[:, :, 0] = CLS_NOP
for _wk, _units in enumerate(_WORKER_UNITS):
    for _j, (_c, _w, _src) in enumerate(_units):
        _TBL[_wk, _j, :3] = [_c, _w, _src]

_GBUF = 80
_ZROWS = 72
_IDXN = 80


def _sc_body(src_hbm, zeros_hbm, tbl_hbm, out_hbm, tbl_s, idx_v, gbuf, sem):
    wid = lax.axis_index("s") * NC + lax.axis_index("c")
    iota16 = lax.iota(jnp.int32, 16)
    z16 = jnp.zeros((LANES,), jnp.float32)
    maxrow = jnp.int32(SEQ_LEN - 1)

    pltpu.sync_copy(tbl_hbm.at[wid], tbl_s)

    def _fill_idx(base, count):
        for k in range(-(-count // LANES)):
            idx_v[pl.ds(k * LANES, LANES)] = jnp.minimum(
                iota16 + (base + k * LANES), maxrow)

    def _g_cls(cls):
        n, do = _G_GEOM[cls]
        glen = -(-n // 8) * 8
        fill = -(-glen // LANES) * LANES

        def _run(w, src):
            _fill_idx(src, fill)
            pltpu.async_copy(src_hbm.at[idx_v.at[pl.ds(0, glen)]],
                             gbuf.at[pl.ds(0, glen)], sem).wait()
            pltpu.sync_copy(gbuf.at[pl.ds(0, n)],
                            out_hbm.at[0, w, pl.ds(do, n), :])

        return _run

    def _c_cls(cls):
        n, do = _G_GEOM[cls]
        glen = -(-n // 8) * 8

        if n % 8:
            def _run(w, src):
                src = pl.multiple_of(src, 8)
                pltpu.sync_copy(src_hbm.at[pl.ds(src, glen), :],
                                gbuf.at[pl.ds(0, glen)])
                pltpu.sync_copy(gbuf.at[pl.ds(0, n)],
                                out_hbm.at[0, w, pl.ds(do, n), :])
        else:
            def _run(w, src):
                src = pl.multiple_of(src, 8)
                pltpu.sync_copy(src_hbm.at[pl.ds(src, n), :],
                                out_hbm.at[0, w, pl.ds(do, n), :])

        return _run

    def _mixed(w, src):
        def _clamp49(k):
            idx_v[pl.ds(k * LANES, LANES)] = jnp.minimum(
                iota16 + (k * LANES), jnp.int32(_KLEN0 - 1))

        for k in range(4):
            _clamp49(k)
        pltpu.async_copy(src_hbm.at[idx_v.at[pl.ds(0, 56)]],
                         gbuf.at[pl.ds(0, 56)], sem).wait()
        for r in range(_KLEN0, 56):
            for cb in range(D // LANES):
                gbuf[r, pl.ds(cb * LANES, LANES)] = z16
        pltpu.sync_copy(gbuf.at[pl.ds(0, 48)],
                        out_hbm.at[0, 0, pl.ds(0, 48), :])
        pltpu.sync_copy(gbuf.at[pl.ds(48, 8)],
                        out_hbm.at[0, 0, pl.ds(48, 8), :])

    def _z_cls(cls):
        n, do = _Z_GEOM[cls]

        def _run(w, src):
            pltpu.sync_copy(zeros_hbm.at[pl.ds(0, n)],
                            out_hbm.at[0, 0, pl.ds(do, n), :])

        return _run

    def _nop(w, src):
        pass

    branches = [_g_cls(CLS_G72A), _g_cls(CLS_G72B), _g_cls(CLS_G56),
                _g_cls(CLS_G3), _mixed, _z_cls(CLS_Z16), _z_cls(CLS_Z72),
                _z_cls(CLS_Z56), _z_cls(CLS_Z3), _nop,
                _c_cls(CLS_G72A), _c_cls(CLS_G72B), _c_cls(CLS_G56),
                _c_cls(CLS_G3)]

    for j in range(NUNIT):
        row = tbl_s[j, pl.ds(0, LANES)]
        lax.switch(row[0], branches, row[1], row[2])


@functools.partial(jax.jit, static_argnames=())
def _sc_copy(src, zeros, tbl):
    mesh = plsc.VectorSubcoreMesh(core_axis_name="c", subcore_axis_name="s")
    return pl.kernel(
        _sc_body,
        out_type=jax.ShapeDtypeStruct((1, NUM_WINDOWS, WIN, D), jnp.float32),
        mesh=mesh,
        scratch_types=[
            pltpu.VMEM((NUNIT, 16), jnp.int32),
            pltpu.VMEM((_IDXN,), jnp.int32),
            pltpu.VMEM((_GBUF, D), jnp.float32),
            pltpu.SemaphoreType.DMA,
        ],
    )(src, zeros, tbl)


def kernel(audio_proj, num_frames):
    del num_frames
    src = audio_proj.reshape(SEQ_LEN, D)
    zeros = jnp.zeros((_ZROWS, D), jnp.float32)
    tbl = jnp.asarray(_TBL)
    sub_sequences = _sc_copy(src, zeros, tbl)
    k_lens = jnp.asarray(_K_LENS, dtype=jnp.int32)
    return sub_sequences, k_lens


# speedup vs baseline: 2.1571x; 2.1571x over previous
import functools

import jax
import jax.numpy as jnp
import numpy as np
from jax import lax
from jax.experimental import pallas as pl
from jax.experimental.pallas import tpu as pltpu
from jax.experimental.pallas import tpu_sc as plsc

SEQ_LEN = 4096
D = 768
LANES = 16
NUM_WINDOWS = 21
WIN = 203
NC, NS = 2, 16
NW = NC * NS
NUNIT = 3


def _window_ranges():
    tokens_per_frame = SEQ_LEN / 81
    half = int(tokens_per_frame * 4 / 2)
    pos = []
    for i in range(NUM_WINDOWS):
        if i == 0:
            pos.append(0)
        else:
            st = tokens_per_frame * ((i - 1) * 4 + 1)
            en = tokens_per_frame * (i * 4 + 1)
            pos.append(int((st + en) / 2) - 1)
    ranges = [[p - half, p + half] for p in pos]
    ranges[0] = [-(half * 2 - ranges[1][0]), ranges[1][0]]
    return ranges


_RANGES = _window_ranges()
_K_LENS = []
for _s, _e in _RANGES:
    _vs, _ve = max(_s, 0), min(_e, SEQ_LEN - 1)
    _K_LENS.append(_ve - _vs + 1 if _vs <= _ve else 0)
_KLEN0 = _K_LENS[0]
_STARTS = [max(_s, 0) for _s, _ in _RANGES]

(CLS_G72A, CLS_G72B, CLS_G56, CLS_G3, CLS_M, CLS_Z16, CLS_Z72, CLS_Z56,
 CLS_Z3, CLS_NOP) = range(10)
_G_GEOM = {CLS_G72A: (72, 0), CLS_G72B: (72, 72), CLS_G56: (56, 144),
           CLS_G3: (3, 200)}
_Z_GEOM = {CLS_Z16: (16, 56), CLS_Z72: (72, 72), CLS_Z56: (56, 144),
           CLS_Z3: (3, 200)}

_UNITS = []
for _w in range(1, NUM_WINDOWS):
    _sw = _STARTS[_w]
    _lin = _sw % 8 == 0
    for _cls, (_n, _do) in _G_GEOM.items():
        _UNITS.append((_cls + (CLS_NOP + 1 if _lin else 0), _w, _sw + _do))
_UNITS.append((CLS_M, 0, 0))
for _cls in (CLS_Z16, CLS_Z72, CLS_Z56, CLS_Z3):
    _UNITS.append((_cls, 0, 0))
assert len(_UNITS) == 85

_COST = {CLS_G72A: 92, CLS_G72B: 92, CLS_G56: 76, CLS_G3: 23, CLS_M: 84,
         CLS_Z16: 36, CLS_Z72: 92, CLS_Z56: 76, CLS_Z3: 23,
         CLS_NOP + 1: 40, CLS_NOP + 2: 40, CLS_NOP + 3: 32, CLS_NOP + 4: 12}

_WORKER_UNITS = [[] for _ in range(NW)]
_LOADS = [0.0] * NW
for _u in sorted(_UNITS, key=lambda u: -_COST[u[0]]):
    _open = [i for i in range(NW) if len(_WORKER_UNITS[i]) < NUNIT]
    _k = min(_open, key=lambda i: (_LOADS[i], len(_WORKER_UNITS[i])))
    _WORKER_UNITS[_k].append(_u)
    _LOADS[_k] += _COST[_u[0]]

_TBL = np.zeros((NW, NUNIT, 16), np.int32)
_TBL[:, :, 0] = CLS_NOP
for _wk, _units in enumerate(_WORKER_UNITS):
    for _j, (_c, _w, _src) in enumerate(_units):
        _TBL[_wk, _j, :3] = [_c, _w, _src]

_GBUF = 80
_ZROWS = 72
_IDXN = 80


def _sc_body(src_hbm, zeros_hbm, tbl_hbm, out_hbm, tbl_s, idx_v, gbuf, sem):
    wid = lax.axis_index("s") * NC + lax.axis_index("c")
    iota16 = lax.iota(jnp.int32, 16)
    z16 = jnp.zeros((LANES,), jnp.float32)
    maxrow = jnp.int32(SEQ_LEN - 1)

    pltpu.sync_copy(tbl_hbm.at[wid], tbl_s)

    def _fill_idx(base, count):
        for k in range(-(-count // LANES)):
            idx_v[pl.ds(k * LANES, LANES)] = jnp.minimum(
                iota16 + (base + k * LANES), maxrow)

    def _g_cls(cls):
        n, do = _G_GEOM[cls]
        glen = -(-n // 8) * 8
        fill = -(-glen // LANES) * LANES

        def _run(w, src):
            _fill_idx(src, fill)
            pltpu.async_copy(src_hbm.at[idx_v.at[pl.ds(0, glen)]],
                             gbuf.at[pl.ds(0, glen)], sem).wait()
            pltpu.sync_copy(gbuf.at[pl.ds(0, n)],
                            out_hbm.at[0, w, pl.ds(do, n), :])

        return _run

    def _c_cls(cls):
        n, do = _G_GEOM[cls]
        glen = -(-n // 8) * 8

        def _run(w, src):
            src = pl.multiple_of(src, 8)
            pltpu.sync_copy(src_hbm.at[pl.ds(src, glen), :],
                            gbuf.at[pl.ds(0, glen)])
            pltpu.sync_copy(gbuf.at[pl.ds(0, n)],
                            out_hbm.at[0, w, pl.ds(do, n), :])

        return _run

    def _mixed(w, src):
        def _clamp49(k):
            idx_v[pl.ds(k * LANES, LANES)] = jnp.minimum(
                iota16 + (k * LANES), jnp.int32(_KLEN0 - 1))

        for k in range(4):
            _clamp49(k)
        pltpu.async_copy(src_hbm.at[idx_v.at[pl.ds(0, 56)]],
                         gbuf.at[pl.ds(0, 56)], sem).wait()
        for r in range(_KLEN0, 56):
            for cb in range(D // LANES):
                gbuf[r, pl.ds(cb * LANES, LANES)] = z16
        pltpu.sync_copy(gbuf.at[pl.ds(0, 48)],
                        out_hbm.at[0, 0, pl.ds(0, 48), :])
        pltpu.sync_copy(gbuf.at[pl.ds(48, 8)],
                        out_hbm.at[0, 0, pl.ds(48, 8), :])

    def _z_cls(cls):
        n, do = _Z_GEOM[cls]
        stage = -(-n // 8) * 8

        def _run(w, src):
            pltpu.sync_copy(zeros_hbm.at[pl.ds(0, stage)],
                            gbuf.at[pl.ds(0, stage)])
            pltpu.sync_copy(gbuf.at[pl.ds(0, n)],
                            out_hbm.at[0, 0, pl.ds(do, n), :])

        return _run

    def _nop(w, src):
        pass

    branches = [_g_cls(CLS_G72A), _g_cls(CLS_G72B), _g_cls(CLS_G56),
                _g_cls(CLS_G3), _mixed, _z_cls(CLS_Z16), _z_cls(CLS_Z72),
                _z_cls(CLS_Z56), _z_cls(CLS_Z3), _nop,
                _c_cls(CLS_G72A), _c_cls(CLS_G72B), _c_cls(CLS_G56),
                _c_cls(CLS_G3)]

    for j in range(NUNIT):
        row = tbl_s[j, pl.ds(0, LANES)]
        lax.switch(row[0], branches, row[1], row[2])


@functools.partial(jax.jit, static_argnames=())
def _sc_copy(src, zeros, tbl):
    mesh = plsc.VectorSubcoreMesh(core_axis_name="c", subcore_axis_name="s")
    return pl.kernel(
        _sc_body,
        out_type=jax.ShapeDtypeStruct((1, NUM_WINDOWS, WIN, D), jnp.float32),
        mesh=mesh,
        scratch_types=[
            pltpu.VMEM((NUNIT, 16), jnp.int32),
            pltpu.VMEM((_IDXN,), jnp.int32),
            pltpu.VMEM((_GBUF, D), jnp.float32),
            pltpu.SemaphoreType.DMA,
        ],
    )(src, zeros, tbl)


def kernel(audio_proj, num_frames):
    del num_frames
    src = audio_proj.reshape(SEQ_LEN, D)
    zeros = jnp.zeros((_ZROWS, D), jnp.float32)
    tbl = jnp.asarray(_TBL)
    sub_sequences = _sc_copy(src, zeros, tbl)
    k_lens = jnp.asarray(_K_LENS, dtype=jnp.int32)
    return sub_sequences, k_lens
